# Initial kernel scaffold; baseline (speedup 1.0000x reference)
import jax
import jax.numpy as jnp
from jax.experimental import pallas as pl


def kernel(features, tables):
    raise NotImplementedError("write your pallas kernel here")


# speedup vs baseline: 1.7849x; 1.7849x over previous
import functools

import jax
import jax.numpy as jnp
from jax import lax
from jax.experimental import pallas as pl
from jax.experimental.pallas import tpu as pltpu
from jax.experimental.pallas import tpu_sc as plsc

_NUM_FEATURES = 26
_CARD = 100000
_DIM = 16
_BATCH = 1024
_SEQ = 50
_TOTAL = _BATCH * _SEQ * _NUM_FEATURES
_NW = 32
_PER_W = _TOTAL // _NW
_CHUNK = 3200
_NCHUNKS = _PER_W // _CHUNK


@functools.partial(
    pl.kernel,
    out_type=jax.ShapeDtypeStruct((_TOTAL, _DIM), jnp.float32),
    mesh=plsc.VectorSubcoreMesh(core_axis_name="c", subcore_axis_name="s"),
    scratch_types=[
        pltpu.VMEM((_CHUNK,), jnp.int32),
        pltpu.VMEM((_CHUNK, _DIM), jnp.float32),
        pltpu.SemaphoreType.DMA,
    ],
    compiler_params=pltpu.CompilerParams(use_tc_tiling_on_sc=False),
)
def _gather_rows(idx_hbm, table_hbm, out_hbm, idx_v, rows_v, sem):
    wid = lax.axis_index("s") * 2 + lax.axis_index("c")
    base0 = wid * _PER_W

    def body(g, carry):
        base = base0 + g * _CHUNK
        pltpu.sync_copy(idx_hbm.at[pl.ds(base, _CHUNK)], idx_v)
        pltpu.async_copy(table_hbm.at[idx_v], rows_v, sem).wait()
        pltpu.sync_copy(rows_v, out_hbm.at[pl.ds(base, _CHUNK)])
        return carry

    lax.fori_loop(0, _NCHUNKS, body, 0)


def kernel(features, tables):
    offsets = jnp.arange(_NUM_FEATURES, dtype=jnp.int32) * _CARD
    flat_idx = (features + offsets).reshape(_TOTAL)
    big_table = tables.reshape(_NUM_FEATURES * _CARD, _DIM)
    rows = _gather_rows(flat_idx, big_table)
    return rows.reshape(_BATCH, _SEQ, _NUM_FEATURES * _DIM)


# speedup vs baseline: 1.7970x; 1.0067x over previous
import functools

import jax
import jax.numpy as jnp
from jax import lax
from jax.experimental import pallas as pl
from jax.experimental.pallas import tpu as pltpu
from jax.experimental.pallas import tpu_sc as plsc

_NUM_FEATURES = 26
_CARD = 100000
_DIM = 16
_BATCH = 1024
_SEQ = 50
_TOTAL = _BATCH * _SEQ * _NUM_FEATURES
_NW = 32
_PER_W = _TOTAL // _NW
_CHUNK = 2600
_NCHUNKS = _PER_W // _CHUNK


@functools.partial(
    pl.kernel,
    out_type=jax.ShapeDtypeStruct((_TOTAL, _DIM), jnp.float32),
    mesh=plsc.VectorSubcoreMesh(core_axis_name="c", subcore_axis_name="s"),
    scratch_types=[
        pltpu.VMEM((_NCHUNKS, _CHUNK), jnp.int32),
        pltpu.VMEM((2, _CHUNK, _DIM), jnp.float32),
        pltpu.SemaphoreType.DMA,
        pltpu.SemaphoreType.DMA,
    ],
    compiler_params=pltpu.CompilerParams(use_tc_tiling_on_sc=False),
)
def _gather_rows(idx_hbm, table_hbm, out_hbm, idx_v, rows_v, sem_g, sem_o):
    wid = lax.axis_index("s") * 2 + lax.axis_index("c")
    base0 = wid * _PER_W

    pltpu.sync_copy(idx_hbm.at[pl.ds(wid * _NCHUNKS, _NCHUNKS)], idx_v)

    def out_slice(g):
        return out_hbm.at[pl.ds(base0 + g * _CHUNK, _CHUNK)]

    def body(k, carry):
        for b in range(2):
            g = 2 * k + b

            @pl.when(g >= 2)
            def _wait_prev():
                pltpu.make_async_copy(rows_v.at[b], out_slice(g - 2), sem_o).wait()

            pltpu.async_copy(table_hbm.at[idx_v.at[g]], rows_v.at[b], sem_g).wait()
            pltpu.async_copy(rows_v.at[b], out_slice(g), sem_o)
        return carry

    lax.fori_loop(0, _NCHUNKS // 2, body, 0)

    pltpu.make_async_copy(rows_v.at[0], out_slice(_NCHUNKS - 2), sem_o).wait()
    pltpu.make_async_copy(rows_v.at[1], out_slice(_NCHUNKS - 1), sem_o).wait()


def kernel(features, tables):
    offsets = jnp.arange(_NUM_FEATURES, dtype=jnp.int32) * _CARD
    flat_idx = (features + offsets).reshape(_NW * _NCHUNKS, _CHUNK)
    big_table = tables.reshape(_NUM_FEATURES * _CARD, _DIM)
    rows = _gather_rows(flat_idx, big_table)
    return rows.reshape(_BATCH, _SEQ, _NUM_FEATURES * _DIM)


# speedup vs baseline: 1.7975x; 1.0003x over previous
import functools

import jax
import jax.numpy as jnp
from jax import lax
from jax.experimental import pallas as pl
from jax.experimental.pallas import tpu as pltpu
from jax.experimental.pallas import tpu_sc as plsc

_NUM_FEATURES = 26
_CARD = 100000
_DIM = 16
_BATCH = 1024
_SEQ = 50
_TOTAL = _BATCH * _SEQ * _NUM_FEATURES
_NW = 32
_PER_W = _TOTAL // _NW
_CHUNK = 2600
_NCHUNKS = _PER_W // _CHUNK


@functools.partial(
    pl.kernel,
    out_type=jax.ShapeDtypeStruct((_TOTAL, _DIM), jnp.float32),
    mesh=plsc.VectorSubcoreMesh(core_axis_name="c", subcore_axis_name="s"),
    scratch_types=[
        pltpu.VMEM((_NCHUNKS, _CHUNK), jnp.int32),
        pltpu.VMEM((2, _CHUNK, _DIM), jnp.float32),
        pltpu.SemaphoreType.DMA,
        pltpu.SemaphoreType.DMA,
    ],
    compiler_params=pltpu.CompilerParams(use_tc_tiling_on_sc=False),
)
def _gather_rows(idx_hbm, table_hbm, out_hbm, idx_v, rows_v, sem_g, sem_o):
    wid = lax.axis_index("s") * 2 + lax.axis_index("c")
    base0 = wid * _PER_W

    pltpu.sync_copy(idx_hbm.at[pl.ds(wid * _NCHUNKS, _NCHUNKS)], idx_v)

    def out_slice(g):
        return out_hbm.at[pl.ds(base0 + g * _CHUNK, _CHUNK)]

    def body(k, carry):
        for b in range(2):
            g = 2 * k + b

            @pl.when(g >= 2)
            def _wait_prev():
                pltpu.make_async_copy(rows_v.at[b], out_slice(g - 2), sem_o).wait()

            pltpu.async_copy(table_hbm.at[idx_v.at[g]], rows_v.at[b], sem_g).wait()
            pltpu.async_copy(rows_v.at[b], out_slice(g), sem_o)
        return carry

    lax.fori_loop(0, _NCHUNKS // 2, body, 0)

    pltpu.make_async_copy(rows_v.at[0], out_slice(_NCHUNKS - 2), sem_o).wait()
    pltpu.make_async_copy(rows_v.at[1], out_slice(_NCHUNKS - 1), sem_o).wait()


def kernel(features, tables):
    offsets = jnp.arange(_NUM_FEATURES, dtype=jnp.int32) * _CARD
    flat_idx = (features + offsets).reshape(_NW * _NCHUNKS, _CHUNK)
    big_table = tables.reshape(_NUM_FEATURES * _CARD * _DIM // 128, 128)
    big_table = jax.lax.optimization_barrier(big_table)
    big_table = big_table.reshape(_NUM_FEATURES * _CARD, _DIM)
    rows = _gather_rows(flat_idx, big_table)
    return rows.reshape(_BATCH, _SEQ, _NUM_FEATURES * _DIM)


# speedup vs baseline: 2.9461x; 1.6390x over previous
import functools

import jax
import jax.numpy as jnp
from jax import lax
from jax.experimental import pallas as pl
from jax.experimental.pallas import tpu as pltpu
from jax.experimental.pallas import tpu_sc as plsc

_NUM_FEATURES = 26
_CARD = 100000
_DIM = 16
_BATCH = 1024
_SEQ = 50
_NCOL = _NUM_FEATURES * _DIM
_NW = 32
_PER_W = _NCOL // _NW
_SCH = 10
_NSC = _SEQ // _SCH
_NTB = _BATCH // 128
_NTC = _NCOL // 8


@functools.partial(
    pl.kernel,
    out_type=jax.ShapeDtypeStruct((_SEQ, _NTC, _NTB, 8, 128), jnp.float32),
    mesh=plsc.VectorSubcoreMesh(core_axis_name="c", subcore_axis_name="s"),
    scratch_types=[
        pltpu.VMEM((_CARD,), jnp.float32),
        pltpu.VMEM((_SCH * _BATCH,), jnp.int32),
        pltpu.VMEM((_SCH, _NTB, 128), jnp.float32),
    ],
    compiler_params=pltpu.CompilerParams(
        use_tc_tiling_on_sc=False, needs_layout_passes=False
    ),
)
def _plane_gather(idx_hbm, planes_hbm, out_hbm, plane_v, idx_v, out_v):
    wid = lax.axis_index("s") * 2 + lax.axis_index("c")
    c0 = wid * _PER_W

    for p in range(_PER_W):
        col = c0 + p
        feat = col // _DIM
        tc = col // 8
        cr = col % 8
        pltpu.sync_copy(planes_hbm.at[col, :], plane_v)

        for sc in range(_NSC):
            pltpu.sync_copy(
                idx_hbm.at[
                    pl.ds(feat * _SEQ * _BATCH + sc * _SCH * _BATCH, _SCH * _BATCH)
                ],
                idx_v,
            )

            def srow(s_loc, carry):
                def btile(jb, carry2):
                    for u in range(8):
                        off = s_loc * _BATCH + jb * 128 + u * 16
                        ivec = idx_v[pl.ds(off, 16)]
                        vals = plsc.load_gather(plane_v, [ivec])
                        out_v[s_loc, jb, pl.ds(u * 16, 16)] = vals
                    return carry2

                return lax.fori_loop(0, _NTB, btile, carry)

            lax.fori_loop(0, _SCH, srow, 0)
            pltpu.sync_copy(
                out_v, out_hbm.at[pl.ds(sc * _SCH, _SCH), tc, :, cr, :]
            )


def kernel(features, tables):
    planes = tables.transpose(0, 2, 1).reshape(_NCOL, _CARD)
    idx = features.transpose(2, 1, 0).reshape(_NUM_FEATURES * _SEQ * _BATCH)
    out5 = _plane_gather(idx, planes)
    return out5.transpose(2, 4, 0, 1, 3).reshape(_BATCH, _SEQ, _NCOL)


# speedup vs baseline: 4.1328x; 1.4028x over previous
import functools

import jax
import jax.numpy as jnp
from jax import lax
from jax.experimental import pallas as pl
from jax.experimental.pallas import tpu as pltpu
from jax.experimental.pallas import tpu_sc as plsc

_NUM_FEATURES = 26
_CARD = 100000
_DIM = 16
_BATCH = 1024
_SEQ = 50
_NCOL = _NUM_FEATURES * _DIM
_NW = 32
_PER_W = _NCOL // _NW
_SCH = 10
_NSC = _SEQ // _SCH
_NTB = _BATCH // 128
_NTC = _NCOL // 8


@functools.partial(
    pl.kernel,
    out_type=jax.ShapeDtypeStruct((_SEQ, _NTC, _NTB, 8, 128), jnp.float32),
    mesh=plsc.VectorSubcoreMesh(core_axis_name="c", subcore_axis_name="s"),
    scratch_types=[
        pltpu.VMEM((_CARD,), jnp.float32),
        pltpu.VMEM((_SCH * _BATCH,), jnp.int32),
        pltpu.VMEM((_SCH, _NTB, 128), jnp.float32),
    ],
    compiler_params=pltpu.CompilerParams(
        use_tc_tiling_on_sc=True, needs_layout_passes=False
    ),
)
def _plane_gather(idx_hbm, planes_hbm, out_hbm, plane_v, idx_v, out_v):
    wid = lax.axis_index("s") * 2 + lax.axis_index("c")
    c0 = wid * _PER_W

    for p in range(_PER_W):
        col = c0 + p
        feat = col // _DIM
        tc = col // 8
        cr = col % 8
        pltpu.sync_copy(planes_hbm.at[col, :], plane_v)

        for sc in range(_NSC):
            pltpu.sync_copy(
                idx_hbm.at[
                    pl.ds(feat * _SEQ * _BATCH + sc * _SCH * _BATCH, _SCH * _BATCH)
                ],
                idx_v,
            )

            def srow(s_loc, carry):
                def btile(jb, carry2):
                    for u in range(8):
                        off = s_loc * _BATCH + jb * 128 + u * 16
                        ivec = idx_v[pl.ds(off, 16)]
                        vals = plsc.load_gather(plane_v, [ivec])
                        out_v[s_loc, jb, pl.ds(u * 16, 16)] = vals
                    return carry2

                return lax.fori_loop(0, _NTB, btile, carry)

            lax.fori_loop(0, _SCH, srow, 0)
            pltpu.sync_copy(
                out_v, out_hbm.at[pl.ds(sc * _SCH, _SCH), tc, :, cr, :]
            )


def kernel(features, tables):
    planes = tables.transpose(0, 2, 1).reshape(_NCOL, _CARD)
    idx = features.transpose(2, 1, 0).reshape(_NUM_FEATURES * _SEQ * _BATCH)
    out5 = _plane_gather(idx, planes)
    return out5.transpose(2, 4, 0, 1, 3).reshape(_BATCH, _SEQ, _NCOL)


# speedup vs baseline: 9.0277x; 2.1844x over previous
import functools

import jax
import jax.numpy as jnp
from jax import lax
from jax.experimental import pallas as pl
from jax.experimental.pallas import tpu as pltpu
from jax.experimental.pallas import tpu_sc as plsc

_NUM_FEATURES = 26
_CARD = 100000
_DIM = 16
_BATCH = 1024
_SEQ = 50
_NCOL = _NUM_FEATURES * _DIM
_NW = 32
_PER_W = _NCOL // _NW
_SCH = 10
_NSC = _SEQ // _SCH
_NTB = _BATCH // 128
_NTC = _NCOL // 8


@functools.partial(
    pl.kernel,
    out_type=jax.ShapeDtypeStruct((_SEQ, _NTC, _NTB, 8, 128), jnp.float32),
    mesh=plsc.VectorSubcoreMesh(core_axis_name="c", subcore_axis_name="s"),
    scratch_types=[
        pltpu.VMEM((_CARD,), jnp.float32),
        pltpu.VMEM((_SCH * _BATCH,), jnp.int32),
        pltpu.VMEM((_SCH, _NTB, 128), jnp.float32),
    ],
    compiler_params=pltpu.CompilerParams(
        use_tc_tiling_on_sc=True, needs_layout_passes=False
    ),
)
def _plane_gather(idx_hbm, planes_hbm, out_hbm, plane_v, idx_v, out_v):
    wid = lax.axis_index("s") * 2 + lax.axis_index("c")
    c0 = wid * _PER_W

    for p in range(_PER_W):
        col = c0 + p
        feat = col // _DIM
        tc = col // 8
        cr = col % 8
        pltpu.sync_copy(planes_hbm.at[col, :], plane_v)

        for sc in range(_NSC):
            pltpu.sync_copy(
                idx_hbm.at[
                    pl.ds(feat * _SEQ * _BATCH + sc * _SCH * _BATCH, _SCH * _BATCH)
                ],
                idx_v,
            )

            @plsc.parallel_loop(0, _SCH * _NTB)
            def _gather_groups(g):
                s_loc = g // _NTB
                jb = g % _NTB
                for u in range(8):
                    off = s_loc * _BATCH + jb * 128 + u * 16
                    ivec = idx_v[pl.ds(off, 16)]
                    vals = plsc.load_gather(plane_v, [ivec])
                    out_v[s_loc, jb, pl.ds(u * 16, 16)] = vals
            pltpu.sync_copy(
                out_v, out_hbm.at[pl.ds(sc * _SCH, _SCH), tc, :, cr, :]
            )


def kernel(features, tables):
    planes = tables.transpose(0, 2, 1).reshape(_NCOL, _CARD)
    idx = features.transpose(2, 1, 0).reshape(_NUM_FEATURES * _SEQ * _BATCH)
    out5 = _plane_gather(idx, planes)
    return out5.transpose(2, 4, 0, 1, 3).reshape(_BATCH, _SEQ, _NCOL)
